# baseline (speedup 1.0000x reference)
import functools

import jax
import jax.numpy as jnp
from jax import lax
from jax.experimental import pallas as pl
from jax.experimental.pallas import tpu as pltpu
from jax.experimental.pallas import tpu_sc as plsc

_B = 16384
_HIDDEN = 36
_HPAD = 48
_NCOMBO = 192

_NC = 2
_NS = 16
_NW = _NC * _NS
_BPW = _B // _NW
_L = 16


def _fused_table_body(st_ref, dt_ref, ht_ref, w_ref, b_ref, out_ref):
    hi = jax.lax.Precision.HIGHEST
    w = w_ref[...]
    sp = jnp.dot(st_ref[...], w[0:12, :], precision=hi)
    dp = jnp.dot(dt_ref[...], w[12:20, :], precision=hi)
    hp = jnp.dot(ht_ref[...], w[20:36, :], precision=hi)

    def onehot(vals, n):
        cols = lax.broadcasted_iota(jnp.int32, (_NCOMBO, n), 1)
        return (vals == cols).astype(jnp.float32)

    rows_s = lax.broadcasted_iota(jnp.int32, (_NCOMBO, 4), 0) // 48
    rows_d = (lax.broadcasted_iota(jnp.int32, (_NCOMBO, 2), 0) // 24) % 2
    rows_h = lax.broadcasted_iota(jnp.int32, (_NCOMBO, 24), 0) % 24
    acc = (jnp.dot(onehot(rows_s, 4), sp, precision=hi)
           + jnp.dot(onehot(rows_d, 2), dp, precision=hi)
           + jnp.dot(onehot(rows_h, 24), hp, precision=hi)
           + b_ref[...])
    out_ref[...] = jnp.concatenate(
        [jnp.maximum(acc, 0.0),
         jnp.zeros((_NCOMBO, _HPAD - _HIDDEN), jnp.float32)], axis=1)


_fused_table = pl.pallas_call(
    _fused_table_body,
    out_shape=jax.ShapeDtypeStruct((_NCOMBO, _HPAD), jnp.float32),
)


@functools.cache
def _make_gather_rows():
    @functools.partial(
        pl.kernel,
        mesh=plsc.VectorSubcoreMesh(core_axis_name="c", subcore_axis_name="s"),
        out_type=jax.ShapeDtypeStruct((_B, _HIDDEN), jnp.float32),
        scratch_types=[
            pltpu.VMEM((_BPW,), jnp.int32),
            pltpu.VMEM((_BPW,), jnp.int32),
            pltpu.VMEM((_BPW,), jnp.int32),
            pltpu.VMEM((_NCOMBO, _HPAD), jnp.float32),
            pltpu.VMEM((_BPW, _HIDDEN), jnp.float32),
            pltpu.SemaphoreType.DMA,
        ],
        compiler_params=pltpu.CompilerParams(use_tc_tiling_on_sc=True,
                                             needs_layout_passes=False),
    )
    def _gather_rows(season_hbm, weekend_hbm, hour_hbm, table_hbm, out_hbm,
                     sv, wv, hv, tv, buf, sem_i):
        wid = lax.axis_index("s") * _NC + lax.axis_index("c")
        base = wid * _BPW
        ci = [pltpu.async_copy(season_hbm.at[pl.ds(base, _BPW)], sv, sem_i),
              pltpu.async_copy(weekend_hbm.at[pl.ds(base, _BPW)], wv, sem_i),
              pltpu.async_copy(hour_hbm.at[pl.ds(base, _BPW)], hv, sem_i)]
        pltpu.sync_copy(table_hbm, tv)
        for c in ci:
            c.wait()

        def body(g, _):
            o = g * _L
            comb = sv[pl.ds(o, _L)] * 48 + wv[pl.ds(o, _L)] * 24 + hv[pl.ds(o, _L)]
            rows = lax.broadcasted_iota(jnp.int32, (_L,), 0) + o
            for c in range(_HIDDEN):
                cvec = jnp.full((_L,), c, jnp.int32)
                plsc.store_scatter(buf, [rows, cvec],
                                   plsc.load_gather(tv, [comb, cvec]))
            return 0

        lax.fori_loop(0, _BPW // _L, body, 0)
        pltpu.sync_copy(buf, out_hbm.at[pl.ds(base, _BPW)])

    return _gather_rows


def kernel(season, is_weekend, hour, season_table, daytype_table, hour_table, W, b):
    table = _fused_table(season_table, daytype_table, hour_table, W,
                         b.reshape(1, _HIDDEN))
    return _make_gather_rows()(season.astype(jnp.int32),
                               is_weekend.astype(jnp.int32),
                               hour.astype(jnp.int32), table)


# speedup vs baseline: 1.3092x; 1.3092x over previous
import functools

import jax
import jax.numpy as jnp
from jax import lax
from jax.experimental import pallas as pl
from jax.experimental.pallas import tpu as pltpu
from jax.experimental.pallas import tpu_sc as plsc

_B = 16384
_HIDDEN = 36
_HPAD = 128
_NCOMBO = 192

_NC = 2
_NS = 16
_NW = _NC * _NS
_BPW = _B // _NW
_L = 16


def _fused_table_body(st_ref, dt_ref, ht_ref, w_ref, b_ref, out_ref):
    hi = jax.lax.Precision.HIGHEST
    w = w_ref[...]
    sp = jnp.dot(st_ref[...], w[0:12, :], precision=hi)
    dp = jnp.dot(dt_ref[...], w[12:20, :], precision=hi)
    hp = jnp.dot(ht_ref[...], w[20:36, :], precision=hi)

    def onehot(vals, n):
        cols = lax.broadcasted_iota(jnp.int32, (_NCOMBO, n), 1)
        return (vals == cols).astype(jnp.float32)

    rows_s = lax.broadcasted_iota(jnp.int32, (_NCOMBO, 4), 0) // 48
    rows_d = (lax.broadcasted_iota(jnp.int32, (_NCOMBO, 2), 0) // 24) % 2
    rows_h = lax.broadcasted_iota(jnp.int32, (_NCOMBO, 24), 0) % 24
    acc = (jnp.dot(onehot(rows_s, 4), sp, precision=hi)
           + jnp.dot(onehot(rows_d, 2), dp, precision=hi)
           + jnp.dot(onehot(rows_h, 24), hp, precision=hi)
           + b_ref[...])
    out_ref[...] = jnp.concatenate(
        [jnp.maximum(acc, 0.0),
         jnp.zeros((_NCOMBO, _HPAD - _HIDDEN), jnp.float32)], axis=1)


_fused_table = pl.pallas_call(
    _fused_table_body,
    out_shape=jax.ShapeDtypeStruct((_NCOMBO, _HPAD), jnp.float32),
)


@functools.cache
def _make_gather_rows():
    @functools.partial(
        pl.kernel,
        mesh=plsc.VectorSubcoreMesh(core_axis_name="c", subcore_axis_name="s"),
        out_type=jax.ShapeDtypeStruct((_B, _HPAD), jnp.float32),
        scratch_types=[
            pltpu.VMEM((_BPW,), jnp.int32),
            pltpu.VMEM((_BPW,), jnp.int32),
            pltpu.VMEM((_BPW,), jnp.int32),
            pltpu.VMEM((_BPW, _HPAD), jnp.float32),
            pltpu.SemaphoreType.DMA,
            pltpu.SemaphoreType.DMA,
        ],
        compiler_params=pltpu.CompilerParams(use_tc_tiling_on_sc=True),
    )
    def _gather_rows(season_hbm, weekend_hbm, hour_hbm, table_hbm, out_hbm,
                     sv, wv, hv, rowsv, sem_i, sem_g):
        wid = lax.axis_index("s") * _NC + lax.axis_index("c")
        base = wid * _BPW
        ci = [pltpu.async_copy(season_hbm.at[pl.ds(base, _BPW)], sv, sem_i),
              pltpu.async_copy(weekend_hbm.at[pl.ds(base, _BPW)], wv, sem_i),
              pltpu.async_copy(hour_hbm.at[pl.ds(base, _BPW)], hv, sem_i)]
        for c in ci:
            c.wait()

        def body(g, _):
            o = g * _L
            sv[pl.ds(o, _L)] = (sv[pl.ds(o, _L)] * 48 + wv[pl.ds(o, _L)] * 24
                                + hv[pl.ds(o, _L)])
            return 0

        lax.fori_loop(0, _BPW // _L, body, 0)
        pltpu.async_copy(table_hbm.at[sv], rowsv, sem_g).wait()
        pltpu.sync_copy(rowsv, out_hbm.at[pl.ds(base, _BPW)])

    return _gather_rows


def kernel(season, is_weekend, hour, season_table, daytype_table, hour_table, W, b):
    table = _fused_table(season_table, daytype_table, hour_table, W,
                         b.reshape(1, _HIDDEN))
    padded = _make_gather_rows()(season.astype(jnp.int32),
                                 is_weekend.astype(jnp.int32),
                                 hour.astype(jnp.int32), table)
    return padded[:, :_HIDDEN]
